# baseline (speedup 1.0000x reference)
import jax
import jax.numpy as jnp
from jax import lax
from jax.experimental import pallas as pl
from jax.experimental.pallas import tpu as pltpu

_BLK = 256


def _som_body(x_ref, w_ref, wout_ref, winner_ref, min_ref, arg_ref):
    j = pl.program_id(0)
    nblk = pl.num_programs(0)
    x = x_ref[...]
    w = w_ref[...]
    xw = lax.dot_general(x, w, (((1,), (1,)), ((), ())),
                         preferred_element_type=jnp.float32)
    wn = jnp.sum(w * w, axis=1, keepdims=True)
    d2 = wn.T - 2.0 * xw
    bmin = jnp.min(d2, axis=1, keepdims=True)
    barg = jnp.argmin(d2, axis=1).astype(jnp.int32)[:, None] + j * _BLK
    wout_ref[...] = w

    @pl.when(j == 0)
    def _init():
        min_ref[...] = bmin
        arg_ref[...] = barg

    @pl.when(j > 0)
    def _merge():
        prev = min_ref[...]
        take = bmin < prev
        min_ref[...] = jnp.where(take, bmin, prev)
        arg_ref[...] = jnp.where(take, barg, arg_ref[...])

    @pl.when(j == nblk - 1)
    def _finish():
        winner_ref[...] = arg_ref[...]


def kernel(inputs, W):
    B, D = inputs.shape
    M, _ = W.shape
    nblk = M // _BLK
    wout, _winner = pl.pallas_call(
        _som_body,
        grid=(nblk,),
        in_specs=[
            pl.BlockSpec((B, D), lambda j: (0, 0)),
            pl.BlockSpec((_BLK, D), lambda j: (j, 0)),
        ],
        out_specs=[
            pl.BlockSpec((_BLK, D), lambda j: (j, 0)),
            pl.BlockSpec((B, 1), lambda j: (0, 0)),
        ],
        out_shape=(
            jax.ShapeDtypeStruct((M, D), W.dtype),
            jax.ShapeDtypeStruct((B, 1), jnp.int32),
        ),
        scratch_shapes=[
            pltpu.VMEM((B, 1), jnp.float32),
            pltpu.VMEM((B, 1), jnp.int32),
        ],
    )(inputs, W)
    return wout


# speedup vs baseline: 2.3180x; 2.3180x over previous
import jax
import jax.numpy as jnp
from jax import lax
from jax.experimental import pallas as pl
from jax.experimental.pallas import tpu as pltpu


def _som_body(x_hbm, w_hbm, wout_hbm, winner_hbm,
              x_v, w_v, win_v, sem_x, sem_w, sem_out, sem_win):
    cp_x = pltpu.make_async_copy(x_hbm, x_v, sem_x)
    cp_w = pltpu.make_async_copy(w_hbm, w_v, sem_w)
    cp_w.start()
    cp_x.start()
    cp_w.wait()
    cp_out = pltpu.make_async_copy(w_v, wout_hbm, sem_out)
    cp_out.start()
    w = w_v[...]
    wn = jnp.sum(w * w, axis=1, keepdims=True)
    cp_x.wait()
    x = x_v[...]
    xw = lax.dot_general(x, w, (((1,), (1,)), ((), ())),
                         preferred_element_type=jnp.float32)
    d2 = wn.T - 2.0 * xw
    win_v[...] = jnp.argmin(d2, axis=1).astype(jnp.int32)[:, None]
    cp_win = pltpu.make_async_copy(win_v, winner_hbm, sem_win)
    cp_win.start()
    cp_win.wait()
    cp_out.wait()


def kernel(inputs, W):
    B, D = inputs.shape
    M, _ = W.shape
    wout, _winner = pl.pallas_call(
        _som_body,
        in_specs=[
            pl.BlockSpec(memory_space=pltpu.MemorySpace.HBM),
            pl.BlockSpec(memory_space=pltpu.MemorySpace.HBM),
        ],
        out_specs=[
            pl.BlockSpec(memory_space=pltpu.MemorySpace.HBM),
            pl.BlockSpec(memory_space=pltpu.MemorySpace.HBM),
        ],
        out_shape=(
            jax.ShapeDtypeStruct((M, D), W.dtype),
            jax.ShapeDtypeStruct((B, 1), jnp.int32),
        ),
        scratch_shapes=[
            pltpu.VMEM((B, D), jnp.float32),
            pltpu.VMEM((M, D), jnp.float32),
            pltpu.VMEM((B, 1), jnp.int32),
            pltpu.SemaphoreType.DMA,
            pltpu.SemaphoreType.DMA,
            pltpu.SemaphoreType.DMA,
            pltpu.SemaphoreType.DMA,
        ],
    )(inputs, W)
    return wout
